# baseline (speedup 1.0000x reference)
import jax
import jax.numpy as jnp
from jax import lax
from jax.experimental import pallas as pl
from jax.experimental.pallas import tpu as pltpu
from jax.experimental.pallas import tpu_sc as plsc

B = 4096
L = 200
EMBED = 64
VOCAB = 1000000
_NBINS = VOCAB // 128
_VPART = _NBINS * 128
_L0 = 112
_L1 = 96

_info = plsc.get_sparse_core_info()
_NC, _NS = _info.num_cores, _info.num_subcores
_NW = _NC * _NS
_SEQ_PER_W = B // _NW

_mesh = plsc.VectorSubcoreMesh(core_axis_name="c", subcore_axis_name="s")
_params = pltpu.CompilerParams(use_tc_tiling_on_sc=True,
                               needs_layout_passes=False)


def _worker_id():
    return lax.axis_index("s") * _NC + lax.axis_index("c")


def _transpose_body(tok_hbm, tail_hbm, stg_hbm, in_vm, tout_vm, isem, osem):
    wid = _worker_id()
    iota = lax.iota(jnp.int32, 16)
    nb = (_NBINS - wid + _NW - 1) // _NW

    def bin_id(k):
        return wid + k * _NW

    def issue_in(k, h):
        pltpu.async_copy(tok_hbm.at[:, pl.ds(bin_id(k) * 128, 128)],
                         in_vm.at[h], isem.at[h])

    def wait_in(k, h):
        pltpu.make_async_copy(tok_hbm.at[:, pl.ds(bin_id(k) * 128, 128)],
                              in_vm.at[h], isem.at[h]).wait()

    def issue_out(k, h):
        pltpu.async_copy(tout_vm.at[h],
                         stg_hbm.at[pl.ds(bin_id(k) * 64, 64)], osem.at[h])

    def wait_out(h):
        pltpu.make_async_copy(tout_vm.at[h], stg_hbm.at[pl.ds(0, 64)],
                              osem.at[h]).wait()

    rows = [tb * 8 + (iota >> 1) for tb in range(8)]
    col0 = (iota & 1) * EMBED

    def transpose(h):
        def jbody(j, c):
            cols = col0 + j
            for tb in range(8):
                v = in_vm[h, j, pl.ds(tb * 16, 16)]
                plsc.store_scatter(tout_vm.at[h], [rows[tb], cols], v)
            return c
        lax.fori_loop(0, EMBED, jbody, 0)

    def step(k, h):
        @pl.when(k + 1 < nb)
        def _():
            issue_in(k + 1, h ^ 1)
        wait_in(k, h)

        @pl.when(k >= 2)
        def _():
            wait_out(h)
        transpose(h)
        issue_out(k, h)

    issue_in(0, 0)

    def pair(g, c):
        k0 = g * 2

        @pl.when(k0 < nb)
        def _():
            step(k0, 0)

        @pl.when(k0 + 1 < nb)
        def _():
            step(k0 + 1, 1)
        return c

    lax.fori_loop(0, (_NBINS + 2 * _NW - 1) // (2 * _NW), pair, 0)
    wait_out(0)
    wait_out(1)

    @pl.when(wid == 0)
    def _():
        pltpu.sync_copy(tail_hbm, tout_vm.at[0, pl.ds(0, 32)])
        pltpu.sync_copy(tout_vm.at[0, pl.ds(0, 32)],
                        stg_hbm.at[pl.ds(_VPART // 2, 32)])


def _gather_body(x_hbm, possp_hbm, stg_hbm, out_hbm,
                 xv, fidx, pbufv, pos_pg, rows_v, out_vm, gsem, psem, osem):
    wid = _worker_id()
    base = wid * _SEQ_PER_W
    iota = lax.iota(jnp.int32, 16)

    pltpu.sync_copy(x_hbm.at[:, pl.ds(base, _SEQ_PER_W)], xv)

    def prep(l, b):
        for bb in range(8):
            v = xv[l, pl.ds(bb * 16, 16)]
            fidx[b, pl.ds(bb * 16, 16)] = v >> 1
            pbufv[b, pl.ds(bb * 16, 16)] = (v & 1) << 6
        pltpu.async_copy(possp_hbm.at[l], pos_pg.at[b], psem.at[b])

    def issue_gather(b):
        pltpu.async_copy(stg_hbm.at[fidx.at[b]], rows_v.at[b], gsem.at[b])

    def wait_gather(b):
        pltpu.make_async_copy(stg_hbm.at[fidx.at[b]], rows_v.at[b],
                              gsem.at[b]).wait()
        pltpu.make_async_copy(possp_hbm.at[0], pos_pg.at[b],
                              psem.at[b]).wait()

    def issue_out(l, b):
        pltpu.async_copy(out_vm.at[b],
                         out_hbm.at[l, :, pl.ds(base, _SEQ_PER_W)],
                         osem.at[b])

    def wait_out(b):
        pltpu.make_async_copy(out_vm.at[b],
                              out_hbm.at[0, :, pl.ds(base, _SEQ_PER_W)],
                              osem.at[b]).wait()

    def compute(l, b):
        pvs = [pbufv[b, pl.ds(bb * 16, 16)] for bb in range(8)]
        bvs = [bb * 16 + iota for bb in range(8)]

        def ebody(e, c):
            pbc = pos_pg[b, pl.ds(e * 16, 16)]
            for bb in range(8):
                col = pvs[bb] + e
                val = plsc.load_gather(rows_v.at[b], [bvs[bb], col])
                out_vm[b, e, pl.ds(bb * 16, 16)] = val + pbc
            return c
        lax.fori_loop(0, EMBED, ebody, 0)

    def halfstep(l, b):
        @pl.when(l + 1 < L)
        def _():
            prep(l + 1, b ^ 1)
            issue_gather(b ^ 1)
        wait_gather(b)

        @pl.when(l >= 2)
        def _():
            wait_out(b)
        compute(l, b)
        issue_out(l, b)

    prep(0, 0)
    issue_gather(0)

    def grp(g, c):
        halfstep(2 * g, 0)
        halfstep(2 * g + 1, 1)
        return c

    lax.fori_loop(0, L // 2, grp, 0)
    wait_out(0)
    wait_out(1)


def kernel(x, tok_table, pos_table):
    xT = x.astype(jnp.int32).T
    tokT = tok_table.T
    pos_splat = jnp.broadcast_to(
        pos_table[:L, :, None], (L, EMBED, 16)).reshape(L, EMBED * 16)
    tail = tok_table[_VPART:].reshape(VOCAB // 2 - _VPART // 2, 128)

    stage1 = pl.kernel(
        _transpose_body,
        mesh=_mesh,
        out_type=jax.ShapeDtypeStruct((VOCAB // 2, 128), jnp.float32),
        scratch_types=[
            pltpu.VMEM((2, EMBED, 128), jnp.float32),
            pltpu.VMEM((2, EMBED, 128), jnp.float32),
            pltpu.SemaphoreType.DMA((2,)),
            pltpu.SemaphoreType.DMA((2,)),
        ],
        compiler_params=_params,
    )
    staging = stage1(tokT, tail)

    stage2 = pl.kernel(
        _gather_body,
        mesh=_mesh,
        out_type=jax.ShapeDtypeStruct((L, EMBED, B), jnp.float32),
        scratch_types=[
            pltpu.VMEM((L, _SEQ_PER_W), jnp.int32),
            pltpu.VMEM((2, _SEQ_PER_W), jnp.int32),
            pltpu.VMEM((2, _SEQ_PER_W), jnp.int32),
            pltpu.VMEM((2, EMBED * 16), jnp.float32),
            pltpu.VMEM((2, _SEQ_PER_W, 128), jnp.float32),
            pltpu.VMEM((2, EMBED, _SEQ_PER_W), jnp.float32),
            pltpu.SemaphoreType.DMA((2,)),
            pltpu.SemaphoreType.DMA((2,)),
            pltpu.SemaphoreType.DMA((2,)),
        ],
        compiler_params=_params,
    )
    outP = stage2(xT, pos_splat, staging)
    return jnp.transpose(outP, (2, 0, 1))


# speedup vs baseline: 1.6865x; 1.6865x over previous
import jax
import jax.numpy as jnp
from jax import lax
from jax.experimental import pallas as pl
from jax.experimental.pallas import tpu as pltpu
from jax.experimental.pallas import tpu_sc as plsc

B = 4096
L = 200
EMBED = 64
VOCAB = 1000000
_NBINS = VOCAB // 128
_VPART = _NBINS * 128
_L0 = 112
_L1 = 96

_info = plsc.get_sparse_core_info()
_NC, _NS = _info.num_cores, _info.num_subcores
_NW = _NC * _NS
_SEQ_PER_W = B // _NW

_mesh = plsc.VectorSubcoreMesh(core_axis_name="c", subcore_axis_name="s")
_params = pltpu.CompilerParams(use_tc_tiling_on_sc=True,
                               needs_layout_passes=False)


def _worker_id():
    return lax.axis_index("s") * _NC + lax.axis_index("c")


def _transpose_body(tok_hbm, tail_hbm, stg_hbm, in_vm, tout_vm, isem, osem):
    wid = _worker_id()
    iota = lax.iota(jnp.int32, 16)
    nb = (_NBINS - wid + _NW - 1) // _NW

    def bin_id(k):
        return wid + k * _NW

    def issue_in(k, h):
        pltpu.async_copy(tok_hbm.at[:, pl.ds(bin_id(k) * 128, 128)],
                         in_vm.at[h], isem.at[h])

    def wait_in(k, h):
        pltpu.make_async_copy(tok_hbm.at[:, pl.ds(bin_id(k) * 128, 128)],
                              in_vm.at[h], isem.at[h]).wait()

    def issue_out(k, h):
        pltpu.async_copy(tout_vm.at[h],
                         stg_hbm.at[pl.ds(bin_id(k) * 64, 64)], osem.at[h])

    def wait_out(h):
        pltpu.make_async_copy(tout_vm.at[h], stg_hbm.at[pl.ds(0, 64)],
                              osem.at[h]).wait()

    rows = [tb * 8 + (iota >> 1) for tb in range(8)]
    col0 = (iota & 1) * EMBED

    def transpose(h):
        @plsc.parallel_loop(0, EMBED, unroll=4)
        def jbody(j):
            cols = col0 + j
            for tb in range(8):
                v = in_vm[h, j, pl.ds(tb * 16, 16)]
                plsc.store_scatter(tout_vm.at[h], [rows[tb], cols], v)

    def step(k, h):
        @pl.when(k + 1 < nb)
        def _():
            issue_in(k + 1, h ^ 1)
        wait_in(k, h)

        @pl.when(k >= 2)
        def _():
            wait_out(h)
        transpose(h)
        issue_out(k, h)

    issue_in(0, 0)

    def pair(g, c):
        k0 = g * 2

        @pl.when(k0 < nb)
        def _():
            step(k0, 0)

        @pl.when(k0 + 1 < nb)
        def _():
            step(k0 + 1, 1)
        return c

    lax.fori_loop(0, (_NBINS + 2 * _NW - 1) // (2 * _NW), pair, 0)
    wait_out(0)
    wait_out(1)

    @pl.when(wid == 0)
    def _():
        pltpu.sync_copy(tail_hbm, tout_vm.at[0, pl.ds(0, 32)])
        pltpu.sync_copy(tout_vm.at[0, pl.ds(0, 32)],
                        stg_hbm.at[pl.ds(_VPART // 2, 32)])


def _gather_body(x_hbm, possp_hbm, stg_hbm, out_hbm,
                 xv, fidx, pbufv, pos_pg, rows_v, out_vm, gsem, psem, osem):
    wid = _worker_id()
    base = wid * _SEQ_PER_W
    iota = lax.iota(jnp.int32, 16)

    pltpu.sync_copy(x_hbm.at[:, pl.ds(base, _SEQ_PER_W)], xv)

    def prep(l, b):
        for bb in range(8):
            v = xv[l, pl.ds(bb * 16, 16)]
            fidx[b, pl.ds(bb * 16, 16)] = v >> 1
            pbufv[b, pl.ds(bb * 16, 16)] = (v & 1) << 6
        pltpu.async_copy(possp_hbm.at[l], pos_pg.at[b], psem.at[b])

    def issue_gather(b):
        pltpu.async_copy(stg_hbm.at[fidx.at[b]], rows_v.at[b], gsem.at[b])

    def wait_gather(b):
        pltpu.make_async_copy(stg_hbm.at[fidx.at[b]], rows_v.at[b],
                              gsem.at[b]).wait()
        pltpu.make_async_copy(possp_hbm.at[0], pos_pg.at[b],
                              psem.at[b]).wait()

    def issue_out(l, b):
        pltpu.async_copy(out_vm.at[b],
                         out_hbm.at[l, :, pl.ds(base, _SEQ_PER_W)],
                         osem.at[b])

    def wait_out(b):
        pltpu.make_async_copy(out_vm.at[b],
                              out_hbm.at[0, :, pl.ds(base, _SEQ_PER_W)],
                              osem.at[b]).wait()

    def compute(l, b):
        pvs = [pbufv[b, pl.ds(bb * 16, 16)] for bb in range(8)]
        bvs = [bb * 16 + iota for bb in range(8)]

        @plsc.parallel_loop(0, EMBED, unroll=4)
        def ebody(e):
            pbc = pos_pg[b, pl.ds(e * 16, 16)]
            for bb in range(8):
                col = pvs[bb] + e
                val = plsc.load_gather(rows_v.at[b], [bvs[bb], col])
                out_vm[b, e, pl.ds(bb * 16, 16)] = val + pbc

    def halfstep(l, b):
        @pl.when(l + 1 < L)
        def _():
            prep(l + 1, b ^ 1)
            issue_gather(b ^ 1)
        wait_gather(b)

        @pl.when(l >= 2)
        def _():
            wait_out(b)
        compute(l, b)
        issue_out(l, b)

    prep(0, 0)
    issue_gather(0)

    def grp(g, c):
        halfstep(2 * g, 0)
        halfstep(2 * g + 1, 1)
        return c

    lax.fori_loop(0, L // 2, grp, 0)
    wait_out(0)
    wait_out(1)


def kernel(x, tok_table, pos_table):
    xT = x.astype(jnp.int32).T
    tokT = tok_table.T
    pos_splat = jnp.broadcast_to(
        pos_table[:L, :, None], (L, EMBED, 16)).reshape(L, EMBED * 16)
    tail = tok_table[_VPART:].reshape(VOCAB // 2 - _VPART // 2, 128)

    stage1 = pl.kernel(
        _transpose_body,
        mesh=_mesh,
        out_type=jax.ShapeDtypeStruct((VOCAB // 2, 128), jnp.float32),
        scratch_types=[
            pltpu.VMEM((2, EMBED, 128), jnp.float32),
            pltpu.VMEM((2, EMBED, 128), jnp.float32),
            pltpu.SemaphoreType.DMA((2,)),
            pltpu.SemaphoreType.DMA((2,)),
        ],
        compiler_params=_params,
    )
    staging = stage1(tokT, tail)

    stage2 = pl.kernel(
        _gather_body,
        mesh=_mesh,
        out_type=jax.ShapeDtypeStruct((L, EMBED, B), jnp.float32),
        scratch_types=[
            pltpu.VMEM((L, _SEQ_PER_W), jnp.int32),
            pltpu.VMEM((2, _SEQ_PER_W), jnp.int32),
            pltpu.VMEM((2, _SEQ_PER_W), jnp.int32),
            pltpu.VMEM((2, EMBED * 16), jnp.float32),
            pltpu.VMEM((2, _SEQ_PER_W, 128), jnp.float32),
            pltpu.VMEM((2, EMBED, _SEQ_PER_W), jnp.float32),
            pltpu.SemaphoreType.DMA((2,)),
            pltpu.SemaphoreType.DMA((2,)),
            pltpu.SemaphoreType.DMA((2,)),
        ],
        compiler_params=_params,
    )
    outP = stage2(xT, pos_splat, staging)
    return jnp.transpose(outP, (2, 0, 1))
